# Initial kernel scaffold; baseline (speedup 1.0000x reference)
import functools

import jax
import jax.numpy as jnp
from jax import lax
from jax.experimental import pallas as pl
from jax.experimental.pallas import tpu as pltpu
from jax.experimental.pallas import tpu_sc as plsc

N = 10000
NP = 10240
DUMMY = 10000
E = 320000
CH = 128
NS = 16
G_CH = 157
E_PAD = G_CH * NS * CH
DH = 128
NG = 64
ROWS_T = 625
RU = 125


def _lin_body(x_ref, w_ref, b_ref, o_ref):
    acc = jnp.dot(x_ref[...], w_ref[0], preferred_element_type=jnp.float32)
    o_ref[...] = jnp.maximum(acc + b_ref[0], 0.0)


def _tc_linear(xp, w_stack, b_stack):
    return pl.pallas_call(
        _lin_body,
        grid=(40,),
        in_specs=[
            pl.BlockSpec((512, 128), lambda i: (i % 20, 0)),
            pl.BlockSpec((1, 128, 128), lambda i: (i // 20, 0, 0)),
            pl.BlockSpec((1, 1, 128), lambda i: (i // 20, 0, 0)),
        ],
        out_specs=pl.BlockSpec((512, 128), lambda i: (i, 0)),
        out_shape=jax.ShapeDtypeStruct((2 * NP, DH), jnp.float32),
    )(xp, w_stack, b_stack)


def _sc_body(h0, src2, dstp, ha, hb, msg, sidx, didx, rows, mbuf, hbuf,
             obuf, zbuf, inv, sem):
    c = lax.axis_index("c")
    s = lax.axis_index("s")

    @pl.loop(0, CH)
    def _(r):
        for k in range(8):
            sl = pl.ds(k * 16, 16)
            zbuf[r, sl] = jnp.zeros((16,), jnp.float32)
            rows[r, sl] = jnp.ones((16,), jnp.float32)

    @pl.loop(0, 5)
    def _(u):
        pltpu.sync_copy(zbuf, msg.at[pl.ds(s * 640 + u * 128, 128)])

    plsc.subcore_barrier()

    @pl.loop(0, G_CH)
    def _(g):
        e0 = (s * G_CH + g) * CH
        pltpu.sync_copy(dstp.at[pl.ds(e0, CH)], didx.at[0])
        pltpu.sync_copy(rows, msg.at[didx.at[0]], add=True)

    plsc.subcore_barrier()

    for u in range(5):
        row0 = s * ROWS_T + u * RU
        pltpu.sync_copy(msg.at[pl.ds(row0, CH)], mbuf)
        for gq in range(8):
            ir = lax.iota(jnp.int32, 16) + gq * 16
            c16 = plsc.load_gather(mbuf, [ir, jnp.zeros((16,), jnp.int32)])
            iv16 = 0.5 / jnp.maximum(c16, 1.0)
            plsc.store_scatter(inv, [ir + u * RU], iv16)

    plsc.subcore_barrier()

    @pl.loop(0, 5)
    def _(u):
        pltpu.sync_copy(zbuf, msg.at[pl.ds(s * 640 + u * 128, 128)])

    plsc.subcore_barrier()

    h_ins = [h0, ha, hb, ha]
    h_outs = [ha, hb, ha, hb]
    for si in range(4):
        hi = h_ins[si]
        ho = h_outs[si]

        @pl.loop(0, G_CH)
        def _(g, hi=hi):
            e0 = (s * G_CH + g) * CH
            pltpu.sync_copy(src2.at[c, pl.ds(e0, CH)], sidx)
            pltpu.sync_copy(dstp.at[pl.ds(e0, CH)], didx.at[0])
            pltpu.async_copy(hi.at[sidx], rows, sem).wait()
            pltpu.sync_copy(rows, msg.at[didx.at[0]], add=True)

        plsc.subcore_barrier()

        for u in range(5):
            row0 = s * ROWS_T + u * RU
            grow0 = c * NP + row0
            pltpu.sync_copy(msg.at[pl.ds(row0, RU)], mbuf.at[pl.ds(0, RU)])
            pltpu.sync_copy(hi.at[pl.ds(grow0, RU)], hbuf.at[pl.ds(0, RU)])

            @pl.loop(0, RU)
            def _(r, u=u):
                iv = inv[u * RU + r]
                for k in range(8):
                    sl = pl.ds(k * 16, 16)
                    obuf[r, sl] = hbuf[r, sl] * 0.5 + mbuf[r, sl] * iv

            pltpu.sync_copy(obuf.at[pl.ds(0, RU)], ho.at[pl.ds(grow0, RU)])
            pltpu.sync_copy(zbuf.at[pl.ds(0, RU)], msg.at[pl.ds(row0, RU)])

        plsc.subcore_barrier()


def _sc_mpnn(h0, src2, dstp):
    mesh = plsc.VectorSubcoreMesh(core_axis_name="c", subcore_axis_name="s")
    f32 = jnp.float32
    kern = pl.kernel(
        _sc_body,
        out_type=[jax.ShapeDtypeStruct((2 * NP, DH), f32),
                  jax.ShapeDtypeStruct((2 * NP, DH), f32)],
        mesh=mesh,
        scratch_types=[
            pltpu.VMEM_SHARED((NP, DH), f32),
            pltpu.VMEM((CH,), jnp.int32),
            pltpu.VMEM((1, CH), jnp.int32),
            pltpu.VMEM((CH, DH), f32),
            pltpu.VMEM((CH, DH), f32),
            pltpu.VMEM((CH, DH), f32),
            pltpu.VMEM((CH, DH), f32),
            pltpu.VMEM((CH, DH), f32),
            pltpu.VMEM((640,), f32),
            pltpu.SemaphoreType.DMA,
        ],
    )
    return kern(h0, src2, dstp)


def _s2s_body(h_ref, bcol_ref, wih_ref, whh_ref, b_ref, wp_ref, bp_ref,
              out_ref):
    f32 = jnp.float32
    bcol = bcol_ref[...]
    valid = bcol < NG
    hm0 = jnp.where(valid, h_ref[:NP], 0.0)
    hm1 = jnp.where(valid, h_ref[NP:], 0.0)
    gid = lax.broadcasted_iota(jnp.int32, (NP, NG), 1)
    P = (bcol == gid).astype(f32)

    q_star = jnp.zeros((NG, 512), f32)
    hl = jnp.zeros((NG, 256), f32)
    cl = jnp.zeros((NG, 256), f32)
    dn00 = (((0,), (0,)), ((), ()))
    dn11 = (((1,), (1,)), ((), ()))
    for _ in range(3):
        gates = (jnp.dot(q_star, wih_ref[...], preferred_element_type=f32)
                 + jnp.dot(hl, whh_ref[...], preferred_element_type=f32)
                 + b_ref[...])
        i_g = jax.nn.sigmoid(gates[:, 0:256])
        f_g = jax.nn.sigmoid(gates[:, 256:512])
        g_g = jnp.tanh(gates[:, 512:768])
        o_g = jax.nn.sigmoid(gates[:, 768:1024])
        cl = f_g * cl + i_g * g_g
        hl = o_g * jnp.tanh(cl)

        qb0 = jnp.dot(P, hl[:, :128], preferred_element_type=f32)
        qb1 = jnp.dot(P, hl[:, 128:], preferred_element_type=f32)
        e = (jnp.sum(hm0 * qb0, axis=1, keepdims=True)
             + jnp.sum(hm1 * qb1, axis=1, keepdims=True))
        em = jnp.where(P > 0, e, -1e30)
        m = jnp.max(em, axis=0, keepdims=True)
        mb = lax.dot_general(P, m, dn11,
                             preferred_element_type=f32)
        a = jnp.exp(e - mb)
        d = lax.dot_general(P, a, dn00,
                            preferred_element_type=f32)
        r0 = lax.dot_general(P, a * hm0, dn00,
                             preferred_element_type=f32)
        r1 = lax.dot_general(P, a * hm1, dn00,
                             preferred_element_type=f32)
        r = jnp.concatenate([r0, r1], axis=1) / jnp.maximum(d, 1e-30)
        q_star = jnp.concatenate([hl, r], axis=1)

    out_ref[...] = (jnp.dot(q_star, wp_ref[...], preferred_element_type=f32)
                    + bp_ref[...])


def _tc_s2s(h4, bcol, wih_t, whh_t, bsum, wp_t, bp):
    return pl.pallas_call(
        _s2s_body,
        out_shape=jax.ShapeDtypeStruct((NG, 1), jnp.float32),
    )(h4, bcol, wih_t, whh_t, bsum, wp_t, bp)


def kernel(x, edge_index, batch, W_in, b_in, W_ih, W_hh, b_ih, b_hh,
           W_pred, b_pred):
    i32 = jnp.int32
    src = edge_index[0].astype(i32)
    dst = edge_index[1].astype(i32)
    srcp = jnp.concatenate([src, jnp.zeros((E_PAD - E,), i32)])
    dstp = jnp.concatenate([dst, jnp.full((E_PAD - E,), DUMMY, i32)])
    src2 = jnp.stack([srcp, srcp + NP])

    xp = jnp.pad(x, ((0, NP - N), (0, 0)))
    w_stack = W_in.reshape(2, 128, 128).transpose(0, 2, 1)
    b_stack = b_in.reshape(2, 1, 128)
    h0 = _tc_linear(xp, w_stack, b_stack)

    _, h4 = _sc_mpnn(h0, src2, dstp)

    bcol = jnp.pad(batch.astype(i32), (0, NP - N),
                   constant_values=NG).reshape(NP, 1)
    wih_t = W_ih.T
    whh_t = W_hh.T
    bsum = (b_ih + b_hh).reshape(1, 4 * 256)
    wp_t = W_pred.T
    bp = b_pred.reshape(1, 1)
    return _tc_s2s(h4, bcol, wih_t, whh_t, bsum, wp_t, bp)


# speedup vs baseline: 3.8354x; 3.8354x over previous
import dataclasses
import functools

import jax
import jax.numpy as jnp
from jax import lax
from jax.experimental import pallas as pl
from jax.experimental.pallas import tpu as pltpu
from jax.experimental.pallas import tpu_sc as plsc

N = 10000
NP = 10240
DUMMY = 10000
E = 320000
CH = 128
NS = 16
G_CH = 157
E_PAD = G_CH * NS * CH
DH = 128
NG = 64
ROWS_T = 640
RU = 128


def _lin_body(x_ref, w_ref, b_ref, o_ref):
    acc = jnp.dot(x_ref[...], w_ref[0], preferred_element_type=jnp.float32)
    o_ref[...] = jnp.maximum(acc + b_ref[0], 0.0)


def _tc_linear(xp, w_stack, b_stack):
    return pl.pallas_call(
        _lin_body,
        grid=(40,),
        in_specs=[
            pl.BlockSpec((512, 128), lambda i: (i % 20, 0)),
            pl.BlockSpec((1, 128, 128), lambda i: (i // 20, 0, 0)),
            pl.BlockSpec((1, 1, 128), lambda i: (i // 20, 0, 0)),
        ],
        out_specs=pl.BlockSpec((512, 128), lambda i: (i, 0)),
        out_shape=jax.ShapeDtypeStruct((2 * NP, DH), jnp.float32),
    )(xp, w_stack, b_stack)


def _sc_body(h0, src2, dstp, ha, hb, msg, sidx, didx, rbuf, hbuf,
             zbuf, inv, sem):
    c = lax.axis_index("c")
    s = lax.axis_index("s")

    @pl.loop(0, CH)
    def _(r):
        for k in range(8):
            sl = pl.ds(k * 16, 16)
            rbuf[r, sl] = jnp.ones((16,), jnp.float32)

    @pl.loop(0, 16)
    def _(r):
        for k in range(8):
            zbuf[r, pl.ds(k * 16, 16)] = jnp.zeros((16,), jnp.float32)

    @pl.loop(0, 40)
    def _(u):
        pltpu.sync_copy(zbuf, msg.at[pl.ds(s * 640 + u * 16, 16)])

    plsc.subcore_barrier()

    @pl.loop(0, G_CH)
    def _(g):
        e0 = (s * G_CH + g) * CH
        pltpu.sync_copy(dstp.at[pl.ds(e0, CH)], didx.at[0])
        pltpu.sync_copy(rbuf, msg.at[didx.at[0]], add=True)

    plsc.subcore_barrier()

    @pl.loop(0, 5)
    def _(u):
        row0 = s * ROWS_T + u * RU
        pltpu.sync_copy(msg.at[pl.ds(row0, CH)], rbuf)

        @pl.loop(0, 8)
        def _(gq, u=u):
            ir = lax.iota(jnp.int32, 16) + gq * 16
            c16 = plsc.load_gather(rbuf, [ir, jnp.zeros((16,), jnp.int32)])
            iv16 = 0.5 / jnp.maximum(c16, 1.0)
            plsc.store_scatter(inv, [ir + u * RU], iv16)

    plsc.subcore_barrier()

    @pl.loop(0, 40)
    def _(u):
        pltpu.sync_copy(zbuf, msg.at[pl.ds(s * 640 + u * 16, 16)])

    plsc.subcore_barrier()

    h_ins = [h0, ha, hb, ha]
    h_outs = [ha, hb, ha, hb]
    for si in range(4):
        hi = h_ins[si]
        ho = h_outs[si]

        @pl.loop(0, G_CH)
        def _(g, hi=hi):
            e0 = (s * G_CH + g) * CH
            pltpu.sync_copy(src2.at[c, pl.ds(e0, CH)], sidx)
            pltpu.sync_copy(dstp.at[pl.ds(e0, CH)], didx.at[0])
            pltpu.async_copy(hi.at[sidx], rbuf, sem).wait()
            pltpu.sync_copy(rbuf, msg.at[didx.at[0]], add=True)

        plsc.subcore_barrier()

        @pl.loop(0, 5)
        def _(u, hi=hi, ho=ho):
            row0 = s * ROWS_T + u * RU
            grow0 = c * NP + row0
            pltpu.sync_copy(msg.at[pl.ds(row0, RU)], rbuf)
            pltpu.sync_copy(hi.at[pl.ds(grow0, RU)], hbuf)

            @pl.loop(0, RU)
            def _(r, u=u):
                ivv = plsc.load_gather(
                    inv, [jnp.full((16,), u * RU + r, jnp.int32)])
                for k in range(8):
                    sl = pl.ds(k * 16, 16)
                    hbuf[r, sl] = hbuf[r, sl] * 0.5 + rbuf[r, sl] * ivv

            pltpu.sync_copy(hbuf, ho.at[pl.ds(grow0, RU)])

            @pl.loop(0, 8)
            def _(j, row0=row0):
                pltpu.sync_copy(zbuf, msg.at[pl.ds(row0 + j * 16, 16)])

        plsc.subcore_barrier()


def _sc_mpnn(h0, src2, dstp):
    mesh = plsc.VectorSubcoreMesh(core_axis_name="c", subcore_axis_name="s")
    f32 = jnp.float32
    cp = pltpu.CompilerParams()
    if "needs_layout_passes" in pltpu.CompilerParams.__dataclass_fields__:
        cp = dataclasses.replace(cp, needs_layout_passes=False)
    kern = pl.kernel(
        _sc_body,
        out_type=[jax.ShapeDtypeStruct((2 * NP, DH), f32),
                  jax.ShapeDtypeStruct((2 * NP, DH), f32)],
        mesh=mesh,
        scratch_types=[
            pltpu.VMEM_SHARED((NP, DH), f32),
            pltpu.VMEM((CH,), jnp.int32),
            pltpu.VMEM((1, CH), jnp.int32),
            pltpu.VMEM((CH, DH), f32),
            pltpu.VMEM((CH, DH), f32),
            pltpu.VMEM((16, DH), f32),
            pltpu.VMEM((640,), f32),
            pltpu.SemaphoreType.DMA,
        ],
        compiler_params=cp,
    )
    return kern(h0, src2, dstp)


def _s2s_body(h_ref, bcol_ref, wih_ref, whh_ref, b_ref, wp_ref, bp_ref,
              out_ref):
    f32 = jnp.float32
    bcol = bcol_ref[...]
    valid = bcol < NG
    hm0 = jnp.where(valid, h_ref[:NP], 0.0)
    hm1 = jnp.where(valid, h_ref[NP:], 0.0)
    gid = lax.broadcasted_iota(jnp.int32, (NP, NG), 1)
    P = (bcol == gid).astype(f32)

    q_star = jnp.zeros((NG, 512), f32)
    hl = jnp.zeros((NG, 256), f32)
    cl = jnp.zeros((NG, 256), f32)
    dn00 = (((0,), (0,)), ((), ()))
    dn11 = (((1,), (1,)), ((), ()))
    for _ in range(3):
        gates = (jnp.dot(q_star, wih_ref[...], preferred_element_type=f32)
                 + jnp.dot(hl, whh_ref[...], preferred_element_type=f32)
                 + b_ref[...])
        i_g = jax.nn.sigmoid(gates[:, 0:256])
        f_g = jax.nn.sigmoid(gates[:, 256:512])
        g_g = jnp.tanh(gates[:, 512:768])
        o_g = jax.nn.sigmoid(gates[:, 768:1024])
        cl = f_g * cl + i_g * g_g
        hl = o_g * jnp.tanh(cl)

        qb0 = jnp.dot(P, hl[:, :128], preferred_element_type=f32)
        qb1 = jnp.dot(P, hl[:, 128:], preferred_element_type=f32)
        e = (jnp.sum(hm0 * qb0, axis=1, keepdims=True)
             + jnp.sum(hm1 * qb1, axis=1, keepdims=True))
        em = jnp.where(P > 0, e, -1e30)
        m = jnp.max(em, axis=0, keepdims=True)
        mb = lax.dot_general(P, m, dn11,
                             preferred_element_type=f32)
        a = jnp.exp(e - mb)
        d = lax.dot_general(P, a, dn00,
                            preferred_element_type=f32)
        r0 = lax.dot_general(P, a * hm0, dn00,
                             preferred_element_type=f32)
        r1 = lax.dot_general(P, a * hm1, dn00,
                             preferred_element_type=f32)
        r = jnp.concatenate([r0, r1], axis=1) / jnp.maximum(d, 1e-30)
        q_star = jnp.concatenate([hl, r], axis=1)

    out_ref[...] = (jnp.dot(q_star, wp_ref[...], preferred_element_type=f32)
                    + bp_ref[...])


def _tc_s2s(h4, bcol, wih_t, whh_t, bsum, wp_t, bp):
    return pl.pallas_call(
        _s2s_body,
        out_shape=jax.ShapeDtypeStruct((NG, 1), jnp.float32),
    )(h4, bcol, wih_t, whh_t, bsum, wp_t, bp)


def kernel(x, edge_index, batch, W_in, b_in, W_ih, W_hh, b_ih, b_hh,
           W_pred, b_pred):
    i32 = jnp.int32
    src = edge_index[0].astype(i32)
    dst = edge_index[1].astype(i32)
    srcp = jnp.concatenate([src, jnp.zeros((E_PAD - E,), i32)])
    dstp = jnp.concatenate([dst, jnp.full((E_PAD - E,), DUMMY, i32)])
    src2 = jnp.stack([srcp, srcp + NP])

    xp = jnp.pad(x, ((0, NP - N), (0, 0)))
    w_stack = W_in.reshape(2, 128, 128).transpose(0, 2, 1)
    b_stack = b_in.reshape(2, 1, 128)
    h0 = _tc_linear(xp, w_stack, b_stack)

    _, h4 = _sc_mpnn(h0, src2, dstp)

    bcol = jnp.pad(batch.astype(i32), (0, NP - N),
                   constant_values=NG).reshape(NP, 1)
    wih_t = W_ih.T
    whh_t = W_hh.T
    bsum = (b_ih + b_hh).reshape(1, 4 * 256)
    wp_t = W_pred.T
    bp = b_pred.reshape(1, 1)
    return _tc_s2s(h4, bcol, wih_t, whh_t, bsum, wp_t, bp)
